# baseline (speedup 1.0000x reference)
import jax
import jax.numpy as jnp
from jax.experimental import pallas as pl

B, T, D = 32, 4096, 80
BINS = 24
PADN = 4112
IMAX = 2**31 - 1


def _energy_kernel(x_ref, o_ref):
    o_ref[...] = jnp.sum(x_ref[...], axis=-1) / jnp.float32(D)


def _shift_right(x, k):
    z = jnp.zeros((x.shape[0], k), x.dtype)
    return jnp.concatenate([z, x[:, :-k]], axis=1)


def _inblock_scan16(x):
    lane = jax.lax.broadcasted_iota(jnp.int32, x.shape, 1) & 15
    acc = x
    for j in range(1, 16):
        acc = acc + jnp.where(lane == j, _shift_right(acc, 1), jnp.float32(0.0))
    return acc


def _dot(a, b):
    return jnp.dot(a, b, precision=jax.lax.Precision.HIGHEST,
                   preferred_element_type=jnp.float32)


def _emulated_cumsum_4112(cin, sel_mats):
    s1, e1m, s2, e2m = sel_mats
    R = cin.shape[0]
    L1 = _inblock_scan16(cin)
    ends1 = _dot(L1, s1)
    e1p = jnp.concatenate(
        [ends1, jnp.zeros((R, 272 - 257), jnp.float32)], axis=1)
    L2 = _inblock_scan16(e1p)
    ends2 = _dot(L2, s2)
    e2p = jnp.concatenate(
        [ends2, jnp.zeros((R, 32 - 17), jnp.float32)], axis=1)
    L3 = _inblock_scan16(e2p)
    off3 = L3[:, 15:16]
    lane32 = jax.lax.broadcasted_iota(jnp.int32, (R, 32), 1)
    off3_full = jnp.where(lane32 < 16, jnp.float32(0.0),
                          jnp.broadcast_to(off3, (R, 32)))
    F3 = L3 + off3_full
    off2 = jnp.concatenate(
        [jnp.zeros((R, 1), jnp.float32), F3[:, :16]], axis=1)
    F2 = L2 + _dot(off2, e2m)
    off1 = jnp.concatenate(
        [jnp.zeros((R, 1), jnp.float32), F2[:, :256]], axis=1)
    F1 = L1 + _dot(off1, e1m)
    return F1


def _main_kernel(energy_ref, uniform_ref, tp_ref,
                 f0_o, f1_o, f2_o, f3_o, f4_o, stats_o):
    f32 = jnp.float32
    energy = energy_ref[...]
    uniform = uniform_ref[...]
    tp = tp_ref[...]

    ones_t = jnp.ones((T, 1), f32)

    def rsum(x):
        return _dot(x, ones_t)

    em = rsum(energy) / f32(T)
    cen = energy - em
    var = rsum(cen * cen) / f32(T - 1)
    es = jnp.maximum(jnp.sqrt(var), f32(1e-6))
    ez = (energy - em) / es

    dif = jnp.abs(energy[:, 1:] - energy[:, :-1])
    delta = jnp.concatenate([jnp.zeros((B, 1), f32), dif], axis=1)

    it_s1 = jax.lax.broadcasted_iota(jnp.int32, (PADN, 257), 0)
    ib_s1 = jax.lax.broadcasted_iota(jnp.int32, (PADN, 257), 1)
    s1 = (it_s1 == 16 * ib_s1 + 15).astype(f32)
    ib_e1 = jax.lax.broadcasted_iota(jnp.int32, (257, PADN), 0)
    it_e1 = jax.lax.broadcasted_iota(jnp.int32, (257, PADN), 1)
    e1m = ((it_e1 >> 4) == ib_e1).astype(f32)
    it_s2 = jax.lax.broadcasted_iota(jnp.int32, (272, 17), 0)
    ib_s2 = jax.lax.broadcasted_iota(jnp.int32, (272, 17), 1)
    s2 = (it_s2 == 16 * ib_s2 + 15).astype(f32)
    ib_e2 = jax.lax.broadcasted_iota(jnp.int32, (17, 272), 0)
    it_e2 = jax.lax.broadcasted_iota(jnp.int32, (17, 272), 1)
    e2m = ((it_e2 >> 4) == ib_e2).astype(f32)
    sel = (s1, e1m, s2, e2m)

    cin5 = jnp.concatenate(
        [jnp.zeros((B, 3), f32), delta, jnp.zeros((B, PADN - T - 3), f32)], axis=1)
    cin7 = jnp.concatenate(
        [jnp.zeros((B, 4), f32), delta, jnp.zeros((B, PADN - T - 4), f32)], axis=1)
    c_all = _emulated_cumsum_4112(
        jnp.concatenate([cin5, cin7], axis=0), sel)
    c5 = c_all[:B]
    c7 = c_all[B:]
    local_rate = (c5[:, 5:4101] - c5[:, :4096]) / f32(5)
    bs = (c7[:, 7:4103] - c7[:, :4096]) / f32(7)

    dbits = jax.lax.bitcast_convert_type(delta, jnp.int32)
    bbits = jax.lax.bitcast_convert_type(bs, jnp.int32)
    bits2 = jnp.concatenate([dbits, bbits], axis=0)
    row2 = jax.lax.broadcasted_iota(jnp.int32, (2 * B, 1), 0)
    kp1 = jnp.where(row2 < B, f32(1434.0), f32(3072.0))

    def body(_, lohi):
        lo, hi = lohi
        mid = lo + (hi - lo) // 2
        cnt = rsum((bits2 <= mid).astype(f32))
        take = cnt >= kp1
        return jnp.where(take, lo, mid + 1), jnp.where(take, mid, hi)

    lo = jnp.zeros((2 * B, 1), jnp.int32)
    hi = jnp.full((2 * B, 1), IMAX)
    lo, hi = jax.lax.fori_loop(0, 31, body, (lo, hi))
    cnt_le = rsum((bits2 <= lo).astype(f32))
    nxt = jnp.min(jnp.where(bits2 > lo, bits2, IMAX), axis=1, keepdims=True)
    hi_bits = jnp.where(cnt_le >= kp1 + f32(1.0), lo, nxt)
    s_lo = jax.lax.bitcast_convert_type(lo, f32)
    s_hi = jax.lax.bitcast_convert_type(hi_bits, f32)
    thr = s_lo * f32(0.75) + s_hi * f32(0.25)
    dthr = thr[:B]
    bthr = thr[B:]

    pause = (ez <= f32(-0.5)) & (delta <= dthr)
    voiced = (ez > f32(-0.1)).astype(f32)
    bev = (bs >= bthr).astype(f32)
    pause_f = pause.astype(f32)

    sp = f32(1.0) - pause_f
    k = 1
    while k < T:
        sp = sp + _shift_right(sp, k)
        k *= 2
    total = jnp.maximum(sp[:, T - 1:T], f32(1.0))
    progress = sp / total
    sdb = progress - uniform

    rights = []
    for j in range(BINS):
        cnt = rsum((progress < tp[:, j:j + 1]).astype(f32))
        rights.append(cnt.astype(jnp.int32))
    right = jnp.concatenate(rights, axis=1)
    left = jnp.clip(right - 1, 0, T - 1)
    r = jnp.clip(right, 0, T - 1)

    feats = (pause_f, local_rate, bev, sdb, voiced)
    iota_g = jax.lax.broadcasted_iota(jnp.int32, (T, 2 * BINS), 0)
    gl = [[] for _ in range(6)]
    gr = [[] for _ in range(6)]
    for b in range(B):
        lr_b = jnp.concatenate([left[b:b + 1], r[b:b + 1]], axis=1)
        oht = (iota_g == lr_b).astype(f32)
        lhs = jnp.concatenate(
            [progress[b:b + 1]] + [fd[b:b + 1] for fd in feats], axis=0)
        g = _dot(lhs, oht)
        for q in range(6):
            gl[q].append(g[q:q + 1, :BINS])
            gr[q].append(g[q:q + 1, BINS:])
    lp = jnp.concatenate(gl[0], axis=0)
    rp = jnp.concatenate(gr[0], axis=0)
    denom = jnp.maximum(jnp.abs(rp - lp), f32(1e-6))
    alpha = jnp.clip((tp - lp) / denom, f32(0.0), f32(1.0))
    lo_edge = right <= 0
    hi_edge = right >= T
    for q, o_ref in enumerate((f0_o, f1_o, f2_o, f3_o, f4_o)):
        v_l = jnp.concatenate(gl[q + 1], axis=0)
        v_r = jnp.concatenate(gr[q + 1], axis=0)
        fd = feats[q]
        val = v_l * (f32(1.0) - alpha) + v_r * alpha
        val = jnp.where(lo_edge, fd[:, 0:1], val)
        val = jnp.where(hi_edge, fd[:, T - 1:T], val)
        o_ref[...] = val

    half = T // 2
    ones_h = jnp.ones((half, 1), f32)
    rate_trend = (_dot(local_rate[:, half:], ones_h) / f32(half)
                  - _dot(local_rate[:, :half], ones_h) / f32(half))

    def run_mean(mask_f):
        prev = _shift_right(mask_f, 1)
        starts = rsum(jnp.where((mask_f > f32(0.5)) & (prev < f32(0.5)),
                                f32(1.0), f32(0.0)))
        tot = rsum(mask_f)
        return tot / jnp.maximum(starts, f32(1.0))

    speech_f = f32(1.0) - pause_f
    stats_o[...] = jnp.concatenate([
        rsum(pause_f) / f32(T),
        run_mean(pause_f),
        run_mean(speech_f),
        rate_trend,
        rsum(bev) / f32(T),
        rsum(voiced) / f32(T),
    ], axis=1)


def kernel(ref_mel):
    ref_mel = ref_mel.astype(jnp.float32)
    energy = pl.pallas_call(
        _energy_kernel,
        grid=(4,),
        in_specs=[pl.BlockSpec((8, T, D), lambda i: (i, 0, 0))],
        out_specs=pl.BlockSpec((8, T), lambda i: (i, 0)),
        out_shape=jax.ShapeDtypeStruct((B, T), jnp.float32),
    )(ref_mel)

    uniform = jnp.linspace(0.0, 1.0, T)[None, :]
    tp = jnp.linspace(0.0, 1.0, BINS)[None, :]

    shapes = [jax.ShapeDtypeStruct((B, BINS), jnp.float32) for _ in range(5)]
    shapes.append(jax.ShapeDtypeStruct((B, 6), jnp.float32))
    f0, f1, f2, f3, f4, stats = pl.pallas_call(
        _main_kernel,
        out_shape=tuple(shapes),
    )(energy, uniform, tp)

    trace = jnp.stack([f0, f1, f2, f3, f4], axis=-1)
    return trace, stats


# speedup vs baseline: 1.4592x; 1.4592x over previous
import jax
import jax.numpy as jnp
from jax.experimental import pallas as pl

B, T, D = 32, 4096, 80
BINS = 24
PADN = 4112
IMAX = 2**31 - 1


def _energy_kernel(x_ref, o_ref):
    o_ref[...] = jnp.sum(x_ref[...], axis=-1) / jnp.float32(D)


def _shift_right(x, k):
    z = jnp.zeros((x.shape[0], k), x.dtype)
    return jnp.concatenate([z, x[:, :-k]], axis=1)


def _inblock_scan16(x):
    lane = jax.lax.broadcasted_iota(jnp.int32, x.shape, 1) & 15
    acc = x
    for j in range(1, 16):
        acc = acc + jnp.where(lane == j, _shift_right(acc, 1), jnp.float32(0.0))
    return acc


def _dot(a, b):
    return jnp.dot(a, b, precision=jax.lax.Precision.HIGHEST,
                   preferred_element_type=jnp.float32)


def _emulated_cumsum_4112(cin, sel_mats):
    s1, e1m, s2, e2m = sel_mats
    R = cin.shape[0]
    L1 = _inblock_scan16(cin)
    ends1 = _dot(L1, s1)
    e1p = jnp.concatenate(
        [ends1, jnp.zeros((R, 272 - 257), jnp.float32)], axis=1)
    L2 = _inblock_scan16(e1p)
    ends2 = _dot(L2, s2)
    e2p = jnp.concatenate(
        [ends2, jnp.zeros((R, 32 - 17), jnp.float32)], axis=1)
    L3 = _inblock_scan16(e2p)
    off3 = L3[:, 15:16]
    lane32 = jax.lax.broadcasted_iota(jnp.int32, (R, 32), 1)
    off3_full = jnp.where(lane32 < 16, jnp.float32(0.0),
                          jnp.broadcast_to(off3, (R, 32)))
    F3 = L3 + off3_full
    off2 = jnp.concatenate(
        [jnp.zeros((R, 1), jnp.float32), F3[:, :16]], axis=1)
    F2 = L2 + _dot(off2, e2m)
    off1 = jnp.concatenate(
        [jnp.zeros((R, 1), jnp.float32), F2[:, :256]], axis=1)
    F1 = L1 + _dot(off1, e1m)
    return F1


def _main_kernel(energy_ref, uniform_ref, tp_ref,
                 f0_o, f1_o, f2_o, f3_o, f4_o, stats_o):
    f32 = jnp.float32
    energy = energy_ref[...]
    uniform = uniform_ref[...]
    tp = tp_ref[...]

    def rsum(x):
        return jnp.sum(x, axis=1, keepdims=True)

    em = rsum(energy) / f32(T)
    cen = energy - em
    var = rsum(cen * cen) / f32(T - 1)
    es = jnp.maximum(jnp.sqrt(var), f32(1e-6))
    ez = (energy - em) / es

    dif = jnp.abs(energy[:, 1:] - energy[:, :-1])
    delta = jnp.concatenate([jnp.zeros((B, 1), f32), dif], axis=1)

    it_s1 = jax.lax.broadcasted_iota(jnp.int32, (PADN, 257), 0)
    ib_s1 = jax.lax.broadcasted_iota(jnp.int32, (PADN, 257), 1)
    s1 = (it_s1 == 16 * ib_s1 + 15).astype(f32)
    ib_e1 = jax.lax.broadcasted_iota(jnp.int32, (257, PADN), 0)
    it_e1 = jax.lax.broadcasted_iota(jnp.int32, (257, PADN), 1)
    e1m = ((it_e1 >> 4) == ib_e1).astype(f32)
    it_s2 = jax.lax.broadcasted_iota(jnp.int32, (272, 17), 0)
    ib_s2 = jax.lax.broadcasted_iota(jnp.int32, (272, 17), 1)
    s2 = (it_s2 == 16 * ib_s2 + 15).astype(f32)
    ib_e2 = jax.lax.broadcasted_iota(jnp.int32, (17, 272), 0)
    it_e2 = jax.lax.broadcasted_iota(jnp.int32, (17, 272), 1)
    e2m = ((it_e2 >> 4) == ib_e2).astype(f32)
    sel = (s1, e1m, s2, e2m)

    cin5 = jnp.concatenate(
        [jnp.zeros((B, 3), f32), delta, jnp.zeros((B, PADN - T - 3), f32)], axis=1)
    cin7 = jnp.concatenate(
        [jnp.zeros((B, 4), f32), delta, jnp.zeros((B, PADN - T - 4), f32)], axis=1)
    c_all = _emulated_cumsum_4112(
        jnp.concatenate([cin5, cin7], axis=0), sel)
    c5 = c_all[:B]
    c7 = c_all[B:]
    local_rate = (c5[:, 5:4101] - c5[:, :4096]) / f32(5)
    bs = (c7[:, 7:4103] - c7[:, :4096]) / f32(7)

    dbits = jax.lax.bitcast_convert_type(delta, jnp.int32)
    bbits = jax.lax.bitcast_convert_type(bs, jnp.int32)
    bits2 = jnp.concatenate([dbits, bbits], axis=0)
    row2 = jax.lax.broadcasted_iota(jnp.int32, (2 * B, 1), 0)
    kp1 = jnp.where(row2 < B, f32(1434.0), f32(3072.0))

    def body(_, lohi):
        lo, hi = lohi
        mid = lo + (hi - lo) // 2
        cnt = rsum((bits2 <= mid).astype(f32))
        take = cnt >= kp1
        return jnp.where(take, lo, mid + 1), jnp.where(take, mid, hi)

    lo = jnp.zeros((2 * B, 1), jnp.int32)
    hi = jnp.full((2 * B, 1), IMAX)
    lo, hi = jax.lax.fori_loop(0, 31, body, (lo, hi))
    cnt_le = rsum((bits2 <= lo).astype(f32))
    nxt = jnp.min(jnp.where(bits2 > lo, bits2, IMAX), axis=1, keepdims=True)
    hi_bits = jnp.where(cnt_le >= kp1 + f32(1.0), lo, nxt)
    s_lo = jax.lax.bitcast_convert_type(lo, f32)
    s_hi = jax.lax.bitcast_convert_type(hi_bits, f32)
    thr = s_lo * f32(0.75) + s_hi * f32(0.25)
    dthr = thr[:B]
    bthr = thr[B:]

    pause = (ez <= f32(-0.5)) & (delta <= dthr)
    voiced = (ez > f32(-0.1)).astype(f32)
    bev = (bs >= bthr).astype(f32)
    pause_f = pause.astype(f32)

    sp = f32(1.0) - pause_f
    k = 1
    while k < T:
        sp = sp + _shift_right(sp, k)
        k *= 2
    total = jnp.maximum(sp[:, T - 1:T], f32(1.0))
    progress = sp / total
    sdb = progress - uniform

    rights = []
    for j in range(BINS):
        cnt = rsum((progress < tp[:, j:j + 1]).astype(f32))
        rights.append(cnt.astype(jnp.int32))
    right = jnp.concatenate(rights, axis=1)
    left = jnp.clip(right - 1, 0, T - 1)
    r = jnp.clip(right, 0, T - 1)

    feats = (pause_f, local_rate, bev, sdb, voiced)
    iota_t = jax.lax.broadcasted_iota(jnp.int32, (B, T), 1)
    outs = [[] for _ in range(5)]
    for j in range(BINS):
        left_j = left[:, j:j + 1]
        r_j = r[:, j:j + 1]
        right_j = right[:, j:j + 1]
        oh_l = (iota_t == left_j).astype(f32)
        oh_r = (iota_t == r_j).astype(f32)
        lp = rsum(oh_l * progress)
        rp = rsum(oh_r * progress)
        denom = jnp.maximum(jnp.abs(rp - lp), f32(1e-6))
        alpha = jnp.clip((tp[:, j:j + 1] - lp) / denom, f32(0.0), f32(1.0))
        lo_edge = right_j <= 0
        hi_edge = right_j >= T
        for q, fd in enumerate(feats):
            v_l = rsum(oh_l * fd)
            v_r = rsum(oh_r * fd)
            val = v_l * (f32(1.0) - alpha) + v_r * alpha
            val = jnp.where(lo_edge, fd[:, 0:1], val)
            val = jnp.where(hi_edge, fd[:, T - 1:T], val)
            outs[q].append(val)
    for q, o_ref in enumerate((f0_o, f1_o, f2_o, f3_o, f4_o)):
        o_ref[...] = jnp.concatenate(outs[q], axis=1)

    half = T // 2
    rate_trend = (rsum(local_rate[:, half:]) / f32(half)
                  - rsum(local_rate[:, :half]) / f32(half))

    def run_mean(mask_f):
        prev = _shift_right(mask_f, 1)
        starts = rsum(jnp.where((mask_f > f32(0.5)) & (prev < f32(0.5)),
                                f32(1.0), f32(0.0)))
        tot = rsum(mask_f)
        return tot / jnp.maximum(starts, f32(1.0))

    speech_f = f32(1.0) - pause_f
    stats_o[...] = jnp.concatenate([
        rsum(pause_f) / f32(T),
        run_mean(pause_f),
        run_mean(speech_f),
        rate_trend,
        rsum(bev) / f32(T),
        rsum(voiced) / f32(T),
    ], axis=1)


def kernel(ref_mel):
    ref_mel = ref_mel.astype(jnp.float32)
    energy = pl.pallas_call(
        _energy_kernel,
        grid=(4,),
        in_specs=[pl.BlockSpec((8, T, D), lambda i: (i, 0, 0))],
        out_specs=pl.BlockSpec((8, T), lambda i: (i, 0)),
        out_shape=jax.ShapeDtypeStruct((B, T), jnp.float32),
    )(ref_mel)

    uniform = jnp.linspace(0.0, 1.0, T)[None, :]
    tp = jnp.linspace(0.0, 1.0, BINS)[None, :]

    shapes = [jax.ShapeDtypeStruct((B, BINS), jnp.float32) for _ in range(5)]
    shapes.append(jax.ShapeDtypeStruct((B, 6), jnp.float32))
    f0, f1, f2, f3, f4, stats = pl.pallas_call(
        _main_kernel,
        out_shape=tuple(shapes),
    )(energy, uniform, tp)

    trace = jnp.stack([f0, f1, f2, f3, f4], axis=-1)
    return trace, stats
